# baseline (speedup 1.0000x reference)
import functools

import jax
import jax.numpy as jnp
from jax import lax
from jax.experimental import pallas as pl
from jax.experimental.pallas import tpu as pltpu
from jax.experimental.pallas import tpu_sc as plsc

N = 10000
E = 320000
D = 128
CW = 16
CH = 128
NW = 32
CHUNKS = E // CH
PER_W = 80
PADC = NW * PER_W + 4
EPAD = PADC * CH
NA = N + 16
PAIRS = PER_W // 2
R = 1000
STRIPE = 640


def _sc_segment_sum(with_counts):
    mesh = plsc.VectorSubcoreMesh(core_axis_name="c", subcore_axis_name="s")
    out_type = [jax.ShapeDtypeStruct((2 * N, D), jnp.float32)]
    scratch = [
        pltpu.VMEM((CH,), jnp.int32),
        pltpu.VMEM((CH,), jnp.int32),
        pltpu.VMEM((CH,), jnp.int32),
        pltpu.VMEM((CH,), jnp.int32),
        pltpu.VMEM((CH, D), jnp.float32),
        pltpu.VMEM((CH, D), jnp.float32),
        pltpu.VMEM((STRIPE,), jnp.float32),
        pltpu.VMEM_SHARED((NA, D), jnp.float32),
        pltpu.SemaphoreType.DMA,
        pltpu.SemaphoreType.DMA,
        pltpu.SemaphoreType.DMA,
        pltpu.SemaphoreType.DMA,
    ]
    if with_counts:
        out_type.append(jax.ShapeDtypeStruct((2 * N,), jnp.float32))
        scratch.insert(8, pltpu.VMEM_SHARED((NA,), jnp.float32))

    def body(*refs):
        if with_counts:
            (src_hbm, dst_hbm, h_hbm, zd_hbm, z16_hbm, ones_hbm,
             acc_out, cnt_out,
             sb0, db0, sb1, db1, rb0, rb1, cbuf, acc_sh, cnt_sh,
             si0, si1, sg0, sg1) = refs
        else:
            (src_hbm, dst_hbm, h_hbm, zd_hbm,
             acc_out,
             sb0, db0, sb1, db1, rb0, rb1, cbuf, acc_sh,
             si0, si1, sg0, sg1) = refs
        cid = lax.axis_index("c")
        sid = lax.axis_index("s")
        wid = sid * 2 + cid
        b0 = sid * STRIPE
        start = wid * PER_W

        pltpu.sync_copy(zd_hbm.at[pl.ds(0, CH)], rb0)
        if with_counts:
            pltpu.sync_copy(z16_hbm.at[pl.ds(0, STRIPE)], cbuf)

        def zero_block(b, n):
            pltpu.sync_copy(rb0.at[pl.ds(0, n)], acc_sh.at[pl.ds(b, n)])

        def zero_cnt(b, n):
            pltpu.sync_copy(cbuf.at[pl.ds(0, n)], cnt_sh.at[pl.ds(b, n)])

        @pl.when(sid < 15)
        def _():
            for k in range(5):
                zero_block(b0 + k * CH, CH)
            if with_counts:
                zero_cnt(b0, STRIPE)

        @pl.when(sid == 15)
        def _():
            for k in range(3):
                zero_block(b0 + k * CH, CH)
            zero_block(b0 + 3 * CH, NA - 15 * STRIPE - 3 * CH)
            if with_counts:
                zero_cnt(b0, NA - 15 * STRIPE)

        if with_counts:
            pltpu.sync_copy(ones_hbm.at[pl.ds(0, STRIPE)], cbuf)
        plsc.subcore_barrier()

        def fire_idx(j, sb, db, si):
            base = (start + j) * CH
            pltpu.async_copy(src_hbm.at[pl.ds(base, CH)], sb, si)
            pltpu.async_copy(dst_hbm.at[pl.ds(base, CH)], db, si)

        def wait_idx(sb, db, si):
            pltpu.make_async_copy(src_hbm.at[pl.ds(0, CH)], sb, si).wait()
            pltpu.make_async_copy(dst_hbm.at[pl.ds(0, CH)], db, si).wait()

        def fire_gather(sb, rb, sg):
            pltpu.async_copy(h_hbm.at[sb], rb, sg)

        def wait_gather(sb, rb, sg):
            pltpu.make_async_copy(h_hbm.at[sb], rb, sg).wait()

        def scat(db, rb):
            pltpu.sync_copy(rb, acc_sh.at[db], add=True)
            if with_counts:
                pltpu.sync_copy(cbuf.at[pl.ds(0, CH)], cnt_sh.at[db],
                                add=True)

        fire_idx(0, sb0, db0, si0)
        fire_idx(1, sb1, db1, si1)
        wait_idx(sb0, db0, si0)
        fire_gather(sb0, rb0, sg0)

        def pair(p, carry):
            a = 2 * p
            wait_gather(sb0, rb0, sg0)
            wait_idx(sb1, db1, si1)
            fire_gather(sb1, rb1, sg1)
            scat(db0, rb0)
            fire_idx(a + 2, sb0, db0, si0)
            wait_gather(sb1, rb1, sg1)
            wait_idx(sb0, db0, si0)
            fire_gather(sb0, rb0, sg0)
            scat(db1, rb1)
            fire_idx(a + 3, sb1, db1, si1)
            return carry

        lax.fori_loop(0, PAIRS, pair, 0)
        wait_gather(sb0, rb0, sg0)
        wait_idx(sb1, db1, si1)
        plsc.subcore_barrier()

        def out_block(b, n):
            pltpu.sync_copy(acc_sh.at[pl.ds(b, n)], rb0.at[pl.ds(0, n)])
            pltpu.sync_copy(rb0.at[pl.ds(0, n)],
                            acc_out.at[pl.ds(cid * N + b, n)])

        def out_cnt(b, n):
            pltpu.sync_copy(cnt_sh.at[pl.ds(b, n)], cbuf.at[pl.ds(0, n)])
            pltpu.sync_copy(cbuf.at[pl.ds(0, n)],
                            cnt_out.at[pl.ds(cid * N + b, n)])

        @pl.when(sid < 15)
        def _():
            for k in range(5):
                out_block(b0 + k * CH, CH)
            if with_counts:
                out_cnt(b0, STRIPE)

        @pl.when(sid == 15)
        def _():
            for k in range(3):
                out_block(b0 + k * CH, CH)
            out_block(b0 + 3 * CH, N - 15 * STRIPE - 3 * CH)
            if with_counts:
                out_cnt(b0, N - 15 * STRIPE)

    return pl.kernel(body, out_type=tuple(out_type), mesh=mesh,
                     scratch_types=scratch)


def _proj_body(x_ref, nk_ref, wp_ref, bp_ref, emb_ref, o_ref):
    nk = nk_ref[...]
    e0 = emb_ref[0:1, :]
    e1 = emb_ref[1:2, :]
    e2 = emb_ref[2:3, :]
    emb = jnp.where(nk == 0, e0, jnp.where(nk == 1, e1, e2))
    h = lax.dot_general(x_ref[...], wp_ref[...], (((1,), (1,)), ((), ())),
                        preferred_element_type=jnp.float32)
    o_ref[...] = h + bp_ref[...] + emb


def _proj(x, nk, wp, bp, emb):
    return pl.pallas_call(
        _proj_body,
        grid=(N // R,),
        in_specs=[
            pl.BlockSpec((R, D), lambda i: (i, 0)),
            pl.BlockSpec((R, 1), lambda i: (i, 0)),
            pl.BlockSpec((D, D), lambda i: (0, 0)),
            pl.BlockSpec((1, D), lambda i: (0, 0)),
            pl.BlockSpec((8, D), lambda i: (0, 0)),
        ],
        out_specs=pl.BlockSpec((R, D), lambda i: (i, 0)),
        out_shape=jax.ShapeDtypeStruct((N, D), jnp.float32),
    )(x, nk, wp, bp, emb)


def _sage_body(norm_relu, acc_ref, cnt0_ref, cnt1_ref, h_ref, wl_ref,
               bl_ref, wr_ref, g_ref, be_ref, o_ref):
    a = acc_ref[0] + acc_ref[1]
    c = cnt0_ref[...] + cnt1_ref[...]
    agg = a / jnp.maximum(c, 1.0)
    t = lax.dot_general(agg, wl_ref[...], (((1,), (1,)), ((), ())),
                        preferred_element_type=jnp.float32)
    t = t + bl_ref[...]
    t = t + lax.dot_general(h_ref[...], wr_ref[...], (((1,), (1,)), ((), ())),
                            preferred_element_type=jnp.float32)
    if norm_relu:
        mu = jnp.mean(t, axis=-1, keepdims=True)
        var = jnp.mean((t - mu) ** 2, axis=-1, keepdims=True)
        t = (t - mu) / jnp.sqrt(var + 1e-5) * g_ref[...] + be_ref[...]
        t = jnp.maximum(t, 0.0)
    o_ref[...] = t


def _sage_linear(acc, cnt0, cnt1, h, wl, bl, wr, g, be, norm_relu):
    return pl.pallas_call(
        functools.partial(_sage_body, norm_relu),
        grid=(N // R,),
        in_specs=[
            pl.BlockSpec((2, R, D), lambda i: (0, i, 0)),
            pl.BlockSpec((R, 1), lambda i: (i, 0)),
            pl.BlockSpec((R, 1), lambda i: (i, 0)),
            pl.BlockSpec((R, D), lambda i: (i, 0)),
            pl.BlockSpec((D, D), lambda i: (0, 0)),
            pl.BlockSpec((1, D), lambda i: (0, 0)),
            pl.BlockSpec((D, D), lambda i: (0, 0)),
            pl.BlockSpec((1, D), lambda i: (0, 0)),
            pl.BlockSpec((1, D), lambda i: (0, 0)),
        ],
        out_specs=pl.BlockSpec((R, D), lambda i: (i, 0)),
        out_shape=jax.ShapeDtypeStruct((N, D), jnp.float32),
    )(acc, cnt0, cnt1, h, wl, bl, wr, g, be)


def kernel(x, edge_index, node_kind, Wp, bp, kind_emb, W1l, b1l, W1r,
           gamma, beta, W2l, b2l, W2r):
    pad = EPAD - E
    srcpad = jnp.concatenate(
        [edge_index[0], jnp.arange(pad, dtype=jnp.int32) % N])
    dstpad = jnp.concatenate(
        [edge_index[1], N + (jnp.arange(pad, dtype=jnp.int32) % 16)])
    nk = node_kind.reshape(N, 1)
    bp2 = bp.reshape(1, D)
    b12 = b1l.reshape(1, D)
    b22 = b2l.reshape(1, D)
    g2 = gamma.reshape(1, D)
    be2 = beta.reshape(1, D)
    emb = jnp.zeros((8, D), jnp.float32).at[:3, :].set(kind_emb)
    zd = jnp.zeros((N, D), jnp.float32)
    z16 = jnp.zeros((N,), jnp.float32)
    ones = jnp.ones((STRIPE,), jnp.float32)

    h0 = _proj(x, nk, Wp, bp2, emb)
    acc1, cnt = _sc_segment_sum(True)(srcpad, dstpad, h0, zd, z16, ones)
    acc1 = acc1.reshape(2, N, D)
    cnt0 = cnt[:N].reshape(N, 1)
    cnt1 = cnt[N:].reshape(N, 1)
    h1 = _sage_linear(acc1, cnt0, cnt1, h0, W1l, b12, W1r, g2, be2, True)
    (acc2,) = _sc_segment_sum(False)(srcpad, dstpad, h1, zd)
    acc2 = acc2.reshape(2, N, D)
    out = _sage_linear(acc2, cnt0, cnt1, h1, W2l, b22, W2r, g2, be2, False)
    return out


# speedup vs baseline: 1.0856x; 1.0856x over previous
import functools

import jax
import jax.numpy as jnp
from jax import lax
from jax.experimental import pallas as pl
from jax.experimental.pallas import tpu as pltpu
from jax.experimental.pallas import tpu_sc as plsc

N = 10000
E = 320000
D = 128
CH = 128
NW = 32
PER_W = 80
PADC = NW * PER_W + 4
EPAD = PADC * CH
NA = N + 16
R = 1000
STRIPE = 640


def _sc_segment_sum(with_counts):
    mesh = plsc.VectorSubcoreMesh(core_axis_name="c", subcore_axis_name="s")
    out_type = [jax.ShapeDtypeStruct((2 * N, D), jnp.float32)]
    scratch = [
        pltpu.VMEM((CH,), jnp.int32),
        pltpu.VMEM((CH,), jnp.int32),
        pltpu.VMEM((CH,), jnp.int32),
        pltpu.VMEM((CH,), jnp.int32),
        pltpu.VMEM((CH,), jnp.int32),
        pltpu.VMEM((CH,), jnp.int32),
        pltpu.VMEM((CH, D), jnp.float32),
        pltpu.VMEM((CH, D), jnp.float32),
        pltpu.VMEM((CH, D), jnp.float32),
        pltpu.VMEM((CH,), jnp.float32),
        pltpu.VMEM_SHARED((NA, D), jnp.float32),
        pltpu.SemaphoreType.DMA,
        pltpu.SemaphoreType.DMA,
        pltpu.SemaphoreType.DMA,
        pltpu.SemaphoreType.DMA,
        pltpu.SemaphoreType.DMA,
        pltpu.SemaphoreType.DMA,
    ]
    if with_counts:
        out_type.append(jax.ShapeDtypeStruct((2 * N,), jnp.float32))
        scratch.insert(11, pltpu.VMEM_SHARED((NA,), jnp.float32))

    def body(*refs):
        if with_counts:
            (src_hbm, dst_hbm, h_hbm, zd_hbm, z16_hbm, ones_hbm,
             acc_out, cnt_out,
             sb0, db0, sb1, db1, sb2, db2,
             rb0, rb1, rb2, cbuf, acc_sh, cnt_sh,
             si0, si1, si2, sg0, sg1, sg2) = refs
        else:
            (src_hbm, dst_hbm, h_hbm, zd_hbm,
             acc_out,
             sb0, db0, sb1, db1, sb2, db2,
             rb0, rb1, rb2, cbuf, acc_sh,
             si0, si1, si2, sg0, sg1, sg2) = refs
        cid = lax.axis_index("c")
        sid = lax.axis_index("s")
        wid = sid * 2 + cid
        b0 = sid * STRIPE
        start = wid * PER_W

        sbs = (sb0, sb1, sb2)
        dbs = (db0, db1, db2)
        rbs = (rb0, rb1, rb2)
        sis = (si0, si1, si2)
        sgs = (sg0, sg1, sg2)

        def fire_idx(j, k):
            base = (start + j) * CH
            pltpu.async_copy(src_hbm.at[pl.ds(base, CH)], sbs[k], sis[k])
            pltpu.async_copy(dst_hbm.at[pl.ds(base, CH)], dbs[k], sis[k])

        def wait_idx(k):
            pltpu.make_async_copy(src_hbm.at[pl.ds(0, CH)], sbs[k],
                                  sis[k]).wait()
            pltpu.make_async_copy(dst_hbm.at[pl.ds(0, CH)], dbs[k],
                                  sis[k]).wait()

        def fire_gather(k):
            pltpu.async_copy(h_hbm.at[sbs[k]], rbs[k], sgs[k])

        def wait_gather(k):
            pltpu.make_async_copy(h_hbm.at[sbs[k]], rbs[k], sgs[k]).wait()

        def scat(j, k):
            @pl.when(start + j < E // CH)
            def _():
                pltpu.sync_copy(rbs[k], acc_sh.at[dbs[k]], add=True)
                if with_counts:
                    pltpu.sync_copy(cbuf.at[pl.ds(0, CH)], cnt_sh.at[dbs[k]],
                                    add=True)

        for k in range(3):
            fire_idx(k, k)

        pltpu.sync_copy(zd_hbm.at[pl.ds(0, CH)], rb0)
        if with_counts:
            pltpu.sync_copy(z16_hbm.at[pl.ds(0, CH)], cbuf)

        def zero_block(b, n):
            pltpu.sync_copy(rb0.at[pl.ds(0, n)], acc_sh.at[pl.ds(b, n)])

        def zero_cnt(b, n):
            pltpu.sync_copy(cbuf.at[pl.ds(0, n)], cnt_sh.at[pl.ds(b, n)])

        @pl.when(sid < 15)
        def _():
            for k in range(5):
                zero_block(b0 + k * CH, CH)
                if with_counts:
                    zero_cnt(b0 + k * CH, CH)

        @pl.when(sid == 15)
        def _():
            for k in range(3):
                zero_block(b0 + k * CH, CH)
                if with_counts:
                    zero_cnt(b0 + k * CH, CH)
            zero_block(b0 + 3 * CH, NA - 15 * STRIPE - 3 * CH)
            if with_counts:
                zero_cnt(b0 + 3 * CH, NA - 15 * STRIPE - 3 * CH)

        if with_counts:
            pltpu.sync_copy(ones_hbm.at[pl.ds(0, CH)], cbuf)
        plsc.subcore_barrier()

        wait_idx(0)
        fire_gather(0)
        wait_idx(1)
        fire_gather(1)

        def triple(p, carry):
            j = 3 * p
            for k in range(3):
                wait_gather(k)
                wait_idx((k + 2) % 3)
                fire_gather((k + 2) % 3)
                scat(j + k, k)
                fire_idx(j + k + 3, k)
            return carry

        lax.fori_loop(0, 26, triple, 0)

        wait_gather(0)
        scat(78, 0)
        wait_gather(1)
        scat(79, 1)
        wait_idx(2)
        plsc.subcore_barrier()

        def stage(b, n, k):
            pltpu.sync_copy(acc_sh.at[pl.ds(b, n)], rbs[k].at[pl.ds(0, n)])
            pltpu.async_copy(rbs[k].at[pl.ds(0, n)],
                             acc_out.at[pl.ds(cid * N + b, n)], sgs[k])

        def stage_wait(b, n, k):
            pltpu.make_async_copy(rbs[k].at[pl.ds(0, n)],
                                  acc_out.at[pl.ds(cid * N + b, n)],
                                  sgs[k]).wait()

        def out_cnt(b, n):
            pltpu.sync_copy(cnt_sh.at[pl.ds(b, n)], cbuf.at[pl.ds(0, n)])
            pltpu.sync_copy(cbuf.at[pl.ds(0, n)],
                            cnt_out.at[pl.ds(cid * N + b, n)])

        @pl.when(sid < 15)
        def _():
            for k in range(5):
                stage(b0 + k * CH, CH, k % 2)
                if k >= 1:
                    stage_wait(b0 + (k - 1) * CH, CH, (k - 1) % 2)
            stage_wait(b0 + 4 * CH, CH, 0)
            if with_counts:
                for k in range(5):
                    out_cnt(b0 + k * CH, CH)

        @pl.when(sid == 15)
        def _():
            for k in range(3):
                stage(b0 + k * CH, CH, k % 2)
                if k >= 1:
                    stage_wait(b0 + (k - 1) * CH, CH, (k - 1) % 2)
            stage(b0 + 3 * CH, N - 15 * STRIPE - 3 * CH, 1)
            stage_wait(b0 + 2 * CH, CH, 0)
            stage_wait(b0 + 3 * CH, N - 15 * STRIPE - 3 * CH, 1)
            if with_counts:
                for k in range(3):
                    out_cnt(b0 + k * CH, CH)
                out_cnt(b0 + 3 * CH, N - 15 * STRIPE - 3 * CH)

    return pl.kernel(body, out_type=tuple(out_type), mesh=mesh,
                     scratch_types=scratch)


def _proj_body(x_ref, nk_ref, wp_ref, bp_ref, emb_ref, o_ref):
    nk = nk_ref[...]
    e0 = emb_ref[0:1, :]
    e1 = emb_ref[1:2, :]
    e2 = emb_ref[2:3, :]
    emb = jnp.where(nk == 0, e0, jnp.where(nk == 1, e1, e2))
    h = lax.dot_general(x_ref[...], wp_ref[...], (((1,), (1,)), ((), ())),
                        preferred_element_type=jnp.float32)
    o_ref[...] = h + bp_ref[...] + emb


def _proj(x, nk, wp, bp, emb):
    return pl.pallas_call(
        _proj_body,
        grid=(N // R,),
        in_specs=[
            pl.BlockSpec((R, D), lambda i: (i, 0)),
            pl.BlockSpec((R, 1), lambda i: (i, 0)),
            pl.BlockSpec((D, D), lambda i: (0, 0)),
            pl.BlockSpec((1, D), lambda i: (0, 0)),
            pl.BlockSpec((8, D), lambda i: (0, 0)),
        ],
        out_specs=pl.BlockSpec((R, D), lambda i: (i, 0)),
        out_shape=jax.ShapeDtypeStruct((N, D), jnp.float32),
    )(x, nk, wp, bp, emb)


def _sage_body(norm_relu, acc_ref, cnt0_ref, cnt1_ref, h_ref, wl_ref,
               bl_ref, wr_ref, g_ref, be_ref, o_ref):
    a = acc_ref[0] + acc_ref[1]
    c = cnt0_ref[...] + cnt1_ref[...]
    agg = a / jnp.maximum(c, 1.0)
    t = lax.dot_general(agg, wl_ref[...], (((1,), (1,)), ((), ())),
                        preferred_element_type=jnp.float32)
    t = t + bl_ref[...]
    t = t + lax.dot_general(h_ref[...], wr_ref[...], (((1,), (1,)), ((), ())),
                            preferred_element_type=jnp.float32)
    if norm_relu:
        mu = jnp.mean(t, axis=-1, keepdims=True)
        var = jnp.mean((t - mu) ** 2, axis=-1, keepdims=True)
        t = (t - mu) / jnp.sqrt(var + 1e-5) * g_ref[...] + be_ref[...]
        t = jnp.maximum(t, 0.0)
    o_ref[...] = t


def _sage_linear(acc, cnt0, cnt1, h, wl, bl, wr, g, be, norm_relu):
    return pl.pallas_call(
        functools.partial(_sage_body, norm_relu),
        grid=(N // R,),
        in_specs=[
            pl.BlockSpec((2, R, D), lambda i: (0, i, 0)),
            pl.BlockSpec((R, 1), lambda i: (i, 0)),
            pl.BlockSpec((R, 1), lambda i: (i, 0)),
            pl.BlockSpec((R, D), lambda i: (i, 0)),
            pl.BlockSpec((D, D), lambda i: (0, 0)),
            pl.BlockSpec((1, D), lambda i: (0, 0)),
            pl.BlockSpec((D, D), lambda i: (0, 0)),
            pl.BlockSpec((1, D), lambda i: (0, 0)),
            pl.BlockSpec((1, D), lambda i: (0, 0)),
        ],
        out_specs=pl.BlockSpec((R, D), lambda i: (i, 0)),
        out_shape=jax.ShapeDtypeStruct((N, D), jnp.float32),
    )(acc, cnt0, cnt1, h, wl, bl, wr, g, be)


def kernel(x, edge_index, node_kind, Wp, bp, kind_emb, W1l, b1l, W1r,
           gamma, beta, W2l, b2l, W2r):
    pad = EPAD - E
    srcpad = jnp.concatenate(
        [edge_index[0], jnp.arange(pad, dtype=jnp.int32) % N])
    dstpad = jnp.concatenate(
        [edge_index[1], N + (jnp.arange(pad, dtype=jnp.int32) % 16)])
    nk = node_kind.reshape(N, 1)
    bp2 = bp.reshape(1, D)
    b12 = b1l.reshape(1, D)
    b22 = b2l.reshape(1, D)
    g2 = gamma.reshape(1, D)
    be2 = beta.reshape(1, D)
    emb = jnp.zeros((8, D), jnp.float32).at[:3, :].set(kind_emb)
    zd = jnp.zeros((CH, D), jnp.float32)
    z16 = jnp.zeros((CH,), jnp.float32)
    ones = jnp.ones((CH,), jnp.float32)

    h0 = _proj(x, nk, Wp, bp2, emb)
    acc1, cnt = _sc_segment_sum(True)(srcpad, dstpad, h0, zd, z16, ones)
    acc1 = acc1.reshape(2, N, D)
    cnt0 = cnt[:N].reshape(N, 1)
    cnt1 = cnt[N:].reshape(N, 1)
    h1 = _sage_linear(acc1, cnt0, cnt1, h0, W1l, b12, W1r, g2, be2, True)
    (acc2,) = _sc_segment_sum(False)(srcpad, dstpad, h1, zd)
    acc2 = acc2.reshape(2, N, D)
    out = _sage_linear(acc2, cnt0, cnt1, h1, W2l, b22, W2r, g2, be2, False)
    return out
